# baseline (speedup 1.0000x reference)
import functools

import jax
import jax.numpy as jnp
from jax import lax
from jax.experimental import pallas as pl
from jax.experimental.pallas import tpu as pltpu

_NUM_POS = 128
_NUM_NEG = 128
_LANES = 128


def _select_kernel(tb_ref, planes_ref, pos_ref, neg_ref, ptgt_ref,
                   *, n_valid, n_rows, n_tgt):
    xc = planes_ref[0]
    yc = planes_ref[1]
    w = planes_ref[2]
    h = planes_ref[3]
    x0 = xc - w / 2
    y0 = yc - h / 2
    x1 = xc + w / 2
    y1 = yc + h / 2
    area_p = (x1 - x0) * (y1 - y0)

    def tgt_body(t, carry):
        miou, targ = carry
        txc = tb_ref[t, 0]
        tyc = tb_ref[t, 1]
        tw = tb_ref[t, 2]
        th = tb_ref[t, 3]
        tx0 = txc - tw / 2
        ty0 = tyc - th / 2
        tx1 = txc + tw / 2
        ty1 = tyc + th / 2
        area_t = (tx1 - tx0) * (ty1 - ty0)
        iw = jnp.maximum(jnp.minimum(x1, tx1) - jnp.maximum(x0, tx0), 0.0)
        ih = jnp.maximum(jnp.minimum(y1, ty1) - jnp.maximum(y0, ty0), 0.0)
        inter = iw * ih
        union = (area_p + area_t) - inter
        iou = inter / jnp.maximum(union, 1e-8)
        upd = iou > miou
        return jnp.where(upd, iou, miou), jnp.where(upd, t, targ)

    miou0 = jnp.full((n_rows, _LANES), -jnp.inf, dtype=jnp.float32)
    targ0 = jnp.zeros((n_rows, _LANES), dtype=jnp.int32)
    miou, targ = lax.fori_loop(0, n_tgt, tgt_body, (miou0, targ0))

    gidx = (lax.broadcasted_iota(jnp.int32, (n_rows, _LANES), 0) * _LANES
            + lax.broadcasted_iota(jnp.int32, (n_rows, _LANES), 1))
    valid = gidx < n_valid
    ninf = jnp.float32(-jnp.inf)
    poskey = jnp.where(valid, miou, ninf)
    negkey = jnp.where(valid, -miou, ninf)
    lane = lax.broadcasted_iota(jnp.int32, (1, _LANES), 1)
    big = jnp.int32(2**30)

    def ext_body(i, c):
        pkey, nkey, pvec, nvec, tvec = c
        pm = jnp.max(pkey)
        pidx = jnp.min(jnp.where(pkey == pm, gidx, big))
        pone = gidx == pidx
        ptgt = jnp.max(jnp.where(pone, targ, -1))
        pkey = jnp.where(pone, ninf, pkey)
        nm = jnp.max(nkey)
        nidx = jnp.min(jnp.where(nkey == nm, gidx, big))
        nkey = jnp.where(gidx == nidx, ninf, nkey)
        sel = lane == i
        pvec = jnp.where(sel, pidx, pvec)
        nvec = jnp.where(sel, nidx, nvec)
        tvec = jnp.where(sel, ptgt, tvec)
        return pkey, nkey, pvec, nvec, tvec

    zero = jnp.zeros((1, _LANES), dtype=jnp.int32)
    _, _, pvec, nvec, tvec = lax.fori_loop(
        0, _NUM_POS, ext_body, (poskey, negkey, zero, zero, zero))
    pos_ref[...] = pvec
    neg_ref[...] = nvec
    ptgt_ref[...] = tvec


def _select_indices(input_boxes, target_boxes):
    b1 = input_boxes.shape[1]
    n_tgt = target_boxes.shape[1]
    n_rows = -(-b1 // _LANES)
    n_rows = -(-n_rows // 8) * 8
    pad = n_rows * _LANES - b1
    planes = jnp.transpose(input_boxes[0])
    planes = jnp.pad(planes, ((0, 0), (0, pad)))
    planes = planes.reshape(4, n_rows, _LANES)
    out_shape = [jax.ShapeDtypeStruct((1, _LANES), jnp.int32)] * 3
    pos, neg, ptgt = pl.pallas_call(
        functools.partial(_select_kernel, n_valid=b1, n_rows=n_rows,
                          n_tgt=n_tgt),
        out_shape=out_shape,
        in_specs=[
            pl.BlockSpec(memory_space=pltpu.SMEM),
            pl.BlockSpec(memory_space=pltpu.VMEM),
        ],
        out_specs=[pl.BlockSpec(memory_space=pltpu.VMEM)] * 3,
    )(target_boxes[0], planes)
    return pos.reshape(-1), neg.reshape(-1), ptgt.reshape(-1)


def kernel(input_boxes, input_anchors, input_trans, input_scores,
           target_boxes, target_labels):
    pos_idx, neg_idx, ptgt_idx = _select_indices(input_boxes, target_boxes)
    out = (
        jnp.take(input_boxes[0], pos_idx, axis=0),
        jnp.take(input_boxes[0], neg_idx, axis=0),
        jnp.take(input_anchors[0], pos_idx, axis=0),
        jnp.take(input_anchors[0], neg_idx, axis=0),
        jnp.take(input_trans[0], pos_idx, axis=0),
        jnp.take(input_trans[0], neg_idx, axis=0),
        jnp.take(input_scores[0], pos_idx, axis=0),
        jnp.take(input_scores[0], neg_idx, axis=0),
        jnp.take(target_boxes[0], ptgt_idx, axis=0),
        jnp.take(target_labels[0], ptgt_idx, axis=0),
    )
    return out


# speedup vs baseline: 1.4767x; 1.4767x over previous
import functools

import jax
import jax.numpy as jnp
from jax import lax
from jax.experimental import pallas as pl
from jax.experimental.pallas import tpu as pltpu

_NUM_POS = 128
_NUM_NEG = 128
_LANES = 128
_ROWS = 160


def _select_kernel(tb_ref, planes_ref, pos_ref, posflat_ref, neg_ref,
                   targ_ref, *, n_valid, n_tgt):
    xc = planes_ref[0]
    yc = planes_ref[1]
    w = planes_ref[2]
    h = planes_ref[3]
    x0 = xc - w / 2
    y0 = yc - h / 2
    x1 = xc + w / 2
    y1 = yc + h / 2
    area_p = (x1 - x0) * (y1 - y0)

    def tgt_body(t, carry):
        miou, targ = carry
        txc = tb_ref[t, 0]
        tyc = tb_ref[t, 1]
        tw = tb_ref[t, 2]
        th = tb_ref[t, 3]
        tx0 = txc - tw / 2
        ty0 = tyc - th / 2
        tx1 = txc + tw / 2
        ty1 = tyc + th / 2
        area_t = (tx1 - tx0) * (ty1 - ty0)
        iw = jnp.maximum(jnp.minimum(x1, tx1) - jnp.maximum(x0, tx0), 0.0)
        ih = jnp.maximum(jnp.minimum(y1, ty1) - jnp.maximum(y0, ty0), 0.0)
        inter = iw * ih
        union = (area_p + area_t) - inter
        iou = inter / jnp.maximum(union, 1e-8)
        upd = iou > miou
        return jnp.where(upd, iou, miou), jnp.where(upd, t, targ)

    miou0 = jnp.full((_ROWS, _LANES), -jnp.inf, dtype=jnp.float32)
    targ0 = jnp.zeros((_ROWS, _LANES), dtype=jnp.int32)
    miou, targ = lax.fori_loop(0, n_tgt, tgt_body, (miou0, targ0))
    targ_ref[...] = targ

    lane = lax.broadcasted_iota(jnp.int32, (1, _LANES), 1)
    row = lax.broadcasted_iota(jnp.int32, (_ROWS, 1), 0)
    gidx = lane * _ROWS + row
    valid = gidx < n_valid
    ninf = jnp.float32(-jnp.inf)
    big = jnp.int32(2**20)

    pkey = jnp.where(valid, miou, ninf)
    nkey = jnp.where(valid, -miou, ninf)

    def col_best(key):
        mx = jnp.max(key, axis=0, keepdims=True)
        rw = jnp.min(jnp.where(key == mx, row, big), axis=0, keepdims=True)
        return mx, rw

    pcmax, pcrow = col_best(pkey)
    ncmax, ncrow = col_best(nkey)

    def extract(key, cmax, crow):
        m = jnp.max(cmax, axis=1, keepdims=True)
        packed = jnp.where(cmax == m, lane * 1024 + crow, big)
        p = jnp.min(packed, axis=1, keepdims=True)
        c = p // 1024
        r = p % 1024
        lanec = lane == c
        key = jnp.where(lanec & (row == r), ninf, key)
        colvals = jnp.where(lanec, key, ninf)
        mx = jnp.max(colvals, axis=0, keepdims=True)
        rw = jnp.min(jnp.where(colvals == mx, row, big), axis=0, keepdims=True)
        cmax = jnp.where(lanec, mx, cmax)
        crow = jnp.where(lanec, rw, crow)
        return key, cmax, crow, c * _ROWS + r, r * _LANES + c

    def ext_body(i, s):
        pkey, pcmax, pcrow, nkey, ncmax, ncrow, pvec, fvec, nvec = s
        pkey, pcmax, pcrow, porig, pflat = extract(pkey, pcmax, pcrow)
        nkey, ncmax, ncrow, norig, _ = extract(nkey, ncmax, ncrow)
        sel = lane == i
        pvec = jnp.where(sel, porig, pvec)
        fvec = jnp.where(sel, pflat, fvec)
        nvec = jnp.where(sel, norig, nvec)
        return pkey, pcmax, pcrow, nkey, ncmax, ncrow, pvec, fvec, nvec

    zero = jnp.zeros((1, _LANES), dtype=jnp.int32)
    s = lax.fori_loop(0, _NUM_POS, ext_body,
                      (pkey, pcmax, pcrow, nkey, ncmax, ncrow,
                       zero, zero, zero))
    pos_ref[...] = s[6]
    posflat_ref[...] = s[7]
    neg_ref[...] = s[8]


def _select_indices(input_boxes, target_boxes):
    b1 = input_boxes.shape[1]
    n_tgt = target_boxes.shape[1]
    npad = _ROWS * _LANES
    planes = jnp.transpose(input_boxes[0])
    planes = jnp.pad(planes, ((0, 0), (0, npad - b1)))
    planes = planes.reshape(4, _LANES, _ROWS).transpose(0, 2, 1)
    idx_shape = jax.ShapeDtypeStruct((1, _LANES), jnp.int32)
    pos, posflat, neg, targ = pl.pallas_call(
        functools.partial(_select_kernel, n_valid=b1, n_tgt=n_tgt),
        out_shape=[idx_shape, idx_shape, idx_shape,
                   jax.ShapeDtypeStruct((_ROWS, _LANES), jnp.int32)],
        in_specs=[
            pl.BlockSpec(memory_space=pltpu.SMEM),
            pl.BlockSpec(memory_space=pltpu.VMEM),
        ],
        out_specs=[pl.BlockSpec(memory_space=pltpu.VMEM)] * 4,
    )(target_boxes[0], planes)
    return pos.reshape(-1), posflat.reshape(-1), neg.reshape(-1), targ


def kernel(input_boxes, input_anchors, input_trans, input_scores,
           target_boxes, target_labels):
    pos_idx, posflat_idx, neg_idx, targ = _select_indices(
        input_boxes, target_boxes)
    ptgt_idx = jnp.take(targ.reshape(-1), posflat_idx, axis=0)
    out = (
        jnp.take(input_boxes[0], pos_idx, axis=0),
        jnp.take(input_boxes[0], neg_idx, axis=0),
        jnp.take(input_anchors[0], pos_idx, axis=0),
        jnp.take(input_anchors[0], neg_idx, axis=0),
        jnp.take(input_trans[0], pos_idx, axis=0),
        jnp.take(input_trans[0], neg_idx, axis=0),
        jnp.take(input_scores[0], pos_idx, axis=0),
        jnp.take(input_scores[0], neg_idx, axis=0),
        jnp.take(target_boxes[0], ptgt_idx, axis=0),
        jnp.take(target_labels[0], ptgt_idx, axis=0),
    )
    return out
